# Initial kernel scaffold; baseline (speedup 1.0000x reference)
import jax
import jax.numpy as jnp
from jax.experimental import pallas as pl


def kernel(x, gamma):
    raise NotImplementedError("write your pallas kernel here")


# speedup vs baseline: 34.2749x; 34.2749x over previous
import functools

import jax
import jax.numpy as jnp
from jax import lax
from jax.experimental import pallas as pl
from jax.experimental.pallas import tpu as pltpu
from jax.experimental.pallas import tpu_sc as plsc

Q_MAX = 127.0
QUANTILE = 0.99
GAMMA_MIN = 0.1
GAMMA_MAX = 10.0

NC = 2
NS = 16
L = 16
NW = NC * NS
NBUCKET = 65536
CHUNK = 8192


def _sc_hist(x_flat, b16):
    n = x_flat.shape[0]
    per_w = n // NW
    n_chunks = per_w // CHUNK
    assert per_w % CHUNK == 0 and n % NW == 0 and n_chunks % 2 == 0
    pass2 = b16 is not None
    mesh = plsc.VectorSubcoreMesh(core_axis_name="c", subcore_axis_name="s")

    scratch = [
        pltpu.VMEM((NBUCKET,), jnp.int32),
        pltpu.VMEM((CHUNK,), jnp.int32),
        pltpu.VMEM((CHUNK,), jnp.int32),
        pltpu.SemaphoreType.DMA,
        pltpu.SemaphoreType.DMA,
    ]
    if pass2:
        scratch.append(pltpu.VMEM((L,), jnp.int32))

    def body(*refs):
        if pass2:
            x_hbm, b_hbm, out_hbm, hist, buf0, buf1, sem0, sem1, bvec = refs
        else:
            x_hbm, out_hbm, hist, buf0, buf1, sem0, sem1 = refs
        bufs = (buf0, buf1)
        sems = (sem0, sem1)
        wid = lax.axis_index("s") * NC + lax.axis_index("c")
        base = pl.multiple_of(wid * per_w, CHUNK)

        zeros = jnp.zeros((L,), jnp.int32)

        @pl.loop(0, NBUCKET // L, unroll=8)
        def _zero(i):
            hist[pl.ds(i * L, L)] = zeros

        if pass2:
            pltpu.sync_copy(b_hbm, bvec)
            bsplat = bvec[...]

        ones = jnp.ones((L,), jnp.int32)

        for j in range(2):
            pltpu.make_async_copy(
                x_hbm.at[pl.ds(base + j * CHUNK, CHUNK)], bufs[j], sems[j]
            ).start()

        @pl.loop(0, n_chunks, step=2)
        def _outer(g):
            for j in range(2):
                gg = g + j
                pltpu.make_async_copy(
                    x_hbm.at[pl.ds(base + gg * CHUNK, CHUNK)], bufs[j], sems[j]
                ).wait()

                @pl.loop(0, CHUNK // L, unroll=8)
                def _inner(i):
                    u = bufs[j][pl.ds(i * L, L)] & 0x7FFFFFFF
                    if pass2:
                        hi = lax.shift_right_logical(u, 16)
                        lo = u & 0xFFFF
                        plsc.addupdate_scatter(
                            hist, [lo], ones, mask=(hi == bsplat)
                        )
                    else:
                        plsc.addupdate_scatter(
                            hist, [lax.shift_right_logical(u, 16)], ones
                        )

                @pl.when(gg + 2 < n_chunks)
                def _next():
                    pltpu.make_async_copy(
                        x_hbm.at[pl.ds(base + (gg + 2) * CHUNK, CHUNK)],
                        bufs[j],
                        sems[j],
                    ).start()

        pltpu.sync_copy(hist, out_hbm.at[wid])

    kern = pl.kernel(
        body,
        out_type=jax.ShapeDtypeStruct((NW, NBUCKET), jnp.int32),
        mesh=mesh,
        scratch_types=scratch,
        compiler_params=pltpu.CompilerParams(needs_layout_passes=False),
    )
    if pass2:
        return kern(x_flat, b16)
    return kern(x_flat)


def _cumulative(h):
    s = jnp.sum(h, axis=0)
    c = s
    sh = 1
    while sh < 128:
        c = c + jnp.concatenate(
            [jnp.zeros((512, sh), jnp.int32), c[:, :-sh]], axis=1
        )
        sh *= 2
    rt = c[:, 127:128]
    e = rt
    sh = 1
    while sh < 512:
        e = e + jnp.concatenate(
            [jnp.zeros((sh, 1), jnp.int32), e[:-sh, :]], axis=0
        )
        sh *= 2
    return (e - rt) + c


def _tc_reduce1(h1, k):
    def body(h_ref, b_ref, r_ref):
        cum = _cumulative(h_ref[...])
        mask = cum <= k
        b = jnp.sum(mask.astype(jnp.int32))
        cum_before = jnp.max(jnp.where(mask, cum, 0))
        b_ref[...] = jnp.full((1, L), b, jnp.int32)
        r_ref[...] = jnp.full((1, 1), k - cum_before, jnp.int32)

    return pl.pallas_call(
        body,
        out_shape=(
            jax.ShapeDtypeStruct((1, L), jnp.int32),
            jax.ShapeDtypeStruct((1, 1), jnp.int32),
        ),
    )(h1)


def _tc_reduce2(h2, b2d, r2d, g2d):
    def body(h_ref, b_ref, r_ref, g_ref, s_ref):
        cum = _cumulative(h_ref[...])
        r = r_ref[0, 0]
        low = jnp.sum((cum <= r).astype(jnp.int32))
        qbits = (b_ref[0, 0] << 16) | low
        q = lax.bitcast_convert_type(qbits, jnp.float32)
        gc = jnp.clip(g_ref[0, 0], GAMMA_MIN, GAMMA_MAX)
        s_ref[...] = jnp.full((1, 1), q * gc / Q_MAX, jnp.float32)

    return pl.pallas_call(
        body,
        out_shape=jax.ShapeDtypeStruct((1, 1), jnp.float32),
    )(h2, b2d, r2d, g2d)


def _tc_quantize(x2d, scale):
    m, w = x2d.shape
    bm = 256

    def body(s_ref, x_ref, o_ref):
        s = s_ref[0, 0]
        si = 1.0 / s
        q = jnp.clip(jnp.round(x_ref[...] * si), -Q_MAX, Q_MAX)
        o_ref[...] = q * s

    return pl.pallas_call(
        body,
        grid=(m // bm,),
        in_specs=[
            pl.BlockSpec(memory_space=pltpu.SMEM),
            pl.BlockSpec((bm, w), lambda i: (i, 0)),
        ],
        out_specs=pl.BlockSpec((bm, w), lambda i: (i, 0)),
        out_shape=jax.ShapeDtypeStruct((m, w), jnp.float32),
    )(scale, x2d)


def kernel(x, gamma):
    n = x.size
    k = round(QUANTILE * n)
    x_flat = x.reshape(-1)
    x_bits = lax.bitcast_convert_type(x_flat, jnp.int32)

    h1 = _sc_hist(x_bits, None)
    b2d, r2d = _tc_reduce1(h1.reshape(NW, 512, 128), k)
    h2 = _sc_hist(x_bits, b2d.reshape(L))
    scale = _tc_reduce2(
        h2.reshape(NW, 512, 128), b2d, r2d, gamma.reshape(1, 1)
    )
    out = _tc_quantize(x_flat.reshape(2048, n // 2048), scale)
    return out.reshape(x.shape)


# speedup vs baseline: 65.6538x; 1.9155x over previous
import functools

import jax
import jax.numpy as jnp
from jax import lax
from jax.experimental import pallas as pl
from jax.experimental.pallas import tpu as pltpu
from jax.experimental.pallas import tpu_sc as plsc

Q_MAX = 127.0
QUANTILE = 0.99
GAMMA_MIN = 0.1
GAMMA_MAX = 10.0

NC = 2
NS = 16
L = 16
NW = NC * NS
NBUCKET = 65536
CHUNK = 8192


def _sc_hist(x_flat, b16):
    n = x_flat.shape[0]
    per_w = n // NW
    n_chunks = per_w // CHUNK
    assert per_w % CHUNK == 0 and n % NW == 0 and n_chunks % 2 == 0
    pass2 = b16 is not None
    mesh = plsc.VectorSubcoreMesh(core_axis_name="c", subcore_axis_name="s")

    scratch = [
        pltpu.VMEM((NBUCKET,), jnp.int32),
        pltpu.VMEM((CHUNK,), jnp.int32),
        pltpu.VMEM((CHUNK,), jnp.int32),
        pltpu.SemaphoreType.DMA,
        pltpu.SemaphoreType.DMA,
    ]
    if pass2:
        scratch.append(pltpu.VMEM((L,), jnp.int32))

    def body(*refs):
        if pass2:
            x_hbm, b_hbm, out_hbm, hist, buf0, buf1, sem0, sem1, bvec = refs
        else:
            x_hbm, out_hbm, hist, buf0, buf1, sem0, sem1 = refs
        bufs = (buf0, buf1)
        sems = (sem0, sem1)
        wid = lax.axis_index("s") * NC + lax.axis_index("c")
        base = pl.multiple_of(wid * per_w, CHUNK)

        zeros = jnp.zeros((L,), jnp.int32)

        @plsc.parallel_loop(0, NBUCKET // L, unroll=8)
        def _zero(i):
            hist[pl.ds(i * L, L)] = zeros

        if pass2:
            pltpu.sync_copy(b_hbm, bvec)
            bsplat = bvec[...]

        ones = jnp.ones((L,), jnp.int32)

        for j in range(2):
            pltpu.make_async_copy(
                x_hbm.at[pl.ds(base + j * CHUNK, CHUNK)], bufs[j], sems[j]
            ).start()

        @pl.loop(0, n_chunks, step=2)
        def _outer(g):
            for j in range(2):
                gg = g + j
                pltpu.make_async_copy(
                    x_hbm.at[pl.ds(base + gg * CHUNK, CHUNK)], bufs[j], sems[j]
                ).wait()

                @plsc.parallel_loop(0, CHUNK // L, unroll=8)
                def _inner(i):
                    u = bufs[j][pl.ds(i * L, L)] & 0x7FFFFFFF
                    if pass2:
                        hi = lax.shift_right_logical(u, 16)
                        lo = u & 0xFFFF
                        plsc.addupdate_scatter(
                            hist, [lo], ones, mask=(hi == bsplat)
                        )
                    else:
                        plsc.addupdate_scatter(
                            hist, [lax.shift_right_logical(u, 16)], ones
                        )

                @pl.when(gg + 2 < n_chunks)
                def _next():
                    pltpu.make_async_copy(
                        x_hbm.at[pl.ds(base + (gg + 2) * CHUNK, CHUNK)],
                        bufs[j],
                        sems[j],
                    ).start()

        pltpu.sync_copy(hist, out_hbm.at[wid])

    kern = pl.kernel(
        body,
        out_type=jax.ShapeDtypeStruct((NW, NBUCKET), jnp.int32),
        mesh=mesh,
        scratch_types=scratch,
        compiler_params=pltpu.CompilerParams(needs_layout_passes=False),
    )
    if pass2:
        return kern(x_flat, b16)
    return kern(x_flat)


def _cumulative(h):
    s = jnp.sum(h, axis=0)
    c = s
    sh = 1
    while sh < 128:
        c = c + jnp.concatenate(
            [jnp.zeros((512, sh), jnp.int32), c[:, :-sh]], axis=1
        )
        sh *= 2
    rt = c[:, 127:128]
    e = rt
    sh = 1
    while sh < 512:
        e = e + jnp.concatenate(
            [jnp.zeros((sh, 1), jnp.int32), e[:-sh, :]], axis=0
        )
        sh *= 2
    return (e - rt) + c


def _tc_reduce1(h1, k):
    def body(h_ref, b_ref, r_ref):
        cum = _cumulative(h_ref[...])
        mask = cum <= k
        b = jnp.sum(mask.astype(jnp.int32))
        cum_before = jnp.max(jnp.where(mask, cum, 0))
        b_ref[...] = jnp.full((1, L), b, jnp.int32)
        r_ref[...] = jnp.full((1, 1), k - cum_before, jnp.int32)

    return pl.pallas_call(
        body,
        out_shape=(
            jax.ShapeDtypeStruct((1, L), jnp.int32),
            jax.ShapeDtypeStruct((1, 1), jnp.int32),
        ),
    )(h1)


def _tc_reduce2(h2, b2d, r2d, g2d):
    def body(h_ref, b_ref, r_ref, g_ref, s_ref):
        cum = _cumulative(h_ref[...])
        r = r_ref[0, 0]
        low = jnp.sum((cum <= r).astype(jnp.int32))
        qbits = (b_ref[0, 0] << 16) | low
        q = lax.bitcast_convert_type(qbits, jnp.float32)
        gc = jnp.clip(g_ref[0, 0], GAMMA_MIN, GAMMA_MAX)
        s_ref[...] = jnp.full((1, 1), q * gc / Q_MAX, jnp.float32)

    return pl.pallas_call(
        body,
        out_shape=jax.ShapeDtypeStruct((1, 1), jnp.float32),
    )(h2, b2d, r2d, g2d)


def _tc_quantize(x2d, scale):
    m, w = x2d.shape
    bm = 256

    def body(s_ref, x_ref, o_ref):
        s = s_ref[0, 0]
        si = 1.0 / s
        q = jnp.clip(jnp.round(x_ref[...] * si), -Q_MAX, Q_MAX)
        o_ref[...] = q * s

    return pl.pallas_call(
        body,
        grid=(m // bm,),
        in_specs=[
            pl.BlockSpec(memory_space=pltpu.SMEM),
            pl.BlockSpec((bm, w), lambda i: (i, 0)),
        ],
        out_specs=pl.BlockSpec((bm, w), lambda i: (i, 0)),
        out_shape=jax.ShapeDtypeStruct((m, w), jnp.float32),
    )(scale, x2d)


def kernel(x, gamma):
    n = x.size
    k = round(QUANTILE * n)
    x_flat = x.reshape(-1)
    x_bits = lax.bitcast_convert_type(x_flat, jnp.int32)

    h1 = _sc_hist(x_bits, None)
    b2d, r2d = _tc_reduce1(h1.reshape(NW, 512, 128), k)
    h2 = _sc_hist(x_bits, b2d.reshape(L))
    scale = _tc_reduce2(
        h2.reshape(NW, 512, 128), b2d, r2d, gamma.reshape(1, 1)
    )
    out = _tc_quantize(x_flat.reshape(2048, n // 2048), scale)
    return out.reshape(x.shape)


# speedup vs baseline: 95.3158x; 1.4518x over previous
import functools

import jax
import jax.numpy as jnp
from jax import lax
from jax.experimental import pallas as pl
from jax.experimental.pallas import tpu as pltpu
from jax.experimental.pallas import tpu_sc as plsc

Q_MAX = 127.0
QUANTILE = 0.99
GAMMA_MIN = 0.1
GAMMA_MAX = 10.0

NC = 2
NS = 16
L = 16
NW = NC * NS
NBUCKET = 65536
CHUNK = 8192


def _sc_hist(x_flat, b16):
    n = x_flat.shape[0]
    per_w = n // NW
    n_chunks = per_w // CHUNK
    assert per_w % CHUNK == 0 and n % NW == 0 and n_chunks % 2 == 0
    pass2 = b16 is not None
    mesh = plsc.VectorSubcoreMesh(core_axis_name="c", subcore_axis_name="s")

    scratch = [
        pltpu.VMEM((NBUCKET,), jnp.int32),
        pltpu.VMEM((CHUNK,), jnp.int32),
        pltpu.VMEM((CHUNK,), jnp.int32),
        pltpu.SemaphoreType.DMA,
        pltpu.SemaphoreType.DMA,
    ]
    if pass2:
        scratch.append(pltpu.VMEM((L,), jnp.int32))

    def body(*refs):
        if pass2:
            x_hbm, b_hbm, out_hbm, hist, buf0, buf1, sem0, sem1, bvec = refs
        else:
            x_hbm, out_hbm, hist, buf0, buf1, sem0, sem1 = refs
        bufs = (buf0, buf1)
        sems = (sem0, sem1)
        wid = lax.axis_index("s") * NC + lax.axis_index("c")
        base = pl.multiple_of(wid * per_w, CHUNK)

        zeros = jnp.zeros((L,), jnp.int32)

        @plsc.parallel_loop(0, NBUCKET // L, unroll=8)
        def _zero(i):
            hist[pl.ds(i * L, L)] = zeros

        if pass2:
            pltpu.sync_copy(b_hbm, bvec)
            bsplat = bvec[...]

        ones = jnp.ones((L,), jnp.int32)

        for j in range(2):
            pltpu.make_async_copy(
                x_hbm.at[pl.ds(base + j * CHUNK, CHUNK)], bufs[j], sems[j]
            ).start()

        @pl.loop(0, n_chunks, step=2)
        def _outer(g):
            for j in range(2):
                gg = g + j
                pltpu.make_async_copy(
                    x_hbm.at[pl.ds(base + gg * CHUNK, CHUNK)], bufs[j], sems[j]
                ).wait()

                @plsc.parallel_loop(0, CHUNK // L, unroll=8)
                def _inner(i):
                    u = bufs[j][pl.ds(i * L, L)] & 0x7FFFFFFF
                    if pass2:
                        hi = lax.shift_right_logical(u, 16)
                        lo = u & 0xFFFF
                        plsc.addupdate_scatter(
                            hist, [lo], ones, mask=(hi == bsplat)
                        )
                    else:
                        plsc.addupdate_scatter(
                            hist, [lax.shift_right_logical(u, 16)], ones
                        )

                @pl.when(gg + 2 < n_chunks)
                def _next():
                    pltpu.make_async_copy(
                        x_hbm.at[pl.ds(base + (gg + 2) * CHUNK, CHUNK)],
                        bufs[j],
                        sems[j],
                    ).start()

        pltpu.sync_copy(hist, out_hbm.at[wid])

    kern = pl.kernel(
        body,
        out_type=jax.ShapeDtypeStruct((NW, NBUCKET), jnp.int32),
        mesh=mesh,
        scratch_types=scratch,
        compiler_params=pltpu.CompilerParams(needs_layout_passes=False),
    )
    if pass2:
        return kern(x_flat, b16)
    return kern(x_flat)


def _cumulative(h):
    s = jnp.sum(h, axis=0)
    c = s
    sh = 1
    while sh < 128:
        c = c + jnp.concatenate(
            [jnp.zeros((512, sh), jnp.int32), c[:, :-sh]], axis=1
        )
        sh *= 2
    rt = c[:, 127:128]
    e = rt
    sh = 1
    while sh < 512:
        e = e + jnp.concatenate(
            [jnp.zeros((sh, 1), jnp.int32), e[:-sh, :]], axis=0
        )
        sh *= 2
    return (e - rt) + c


def _tc_reduce1(h1, k):
    def body(h_ref, b_ref, r_ref):
        cum = _cumulative(h_ref[...])
        mask = cum <= k
        b = jnp.sum(mask.astype(jnp.int32))
        cum_before = jnp.max(jnp.where(mask, cum, 0))
        b_ref[...] = jnp.full((1, L), b, jnp.int32)
        r_ref[...] = jnp.full((1, 1), k - cum_before, jnp.int32)

    return pl.pallas_call(
        body,
        out_shape=(
            jax.ShapeDtypeStruct((1, L), jnp.int32),
            jax.ShapeDtypeStruct((1, 1), jnp.int32),
        ),
    )(h1)


def _tc_reduce2(h2, b2d, r2d, g2d):
    def body(h_ref, b_ref, r_ref, g_ref, s_ref):
        cum = _cumulative(h_ref[...])
        r = r_ref[0, 0]
        low = jnp.sum((cum <= r).astype(jnp.int32))
        qbits = (b_ref[0, 0] << 16) | low
        q = lax.bitcast_convert_type(qbits, jnp.float32)
        gc = jnp.clip(g_ref[0, 0], GAMMA_MIN, GAMMA_MAX)
        s_ref[...] = jnp.full((1, 1), q * gc / Q_MAX, jnp.float32)

    return pl.pallas_call(
        body,
        out_shape=jax.ShapeDtypeStruct((1, 1), jnp.float32),
    )(h2, b2d, r2d, g2d)


def _tc_quantize(x3d, scale):
    a, m, w = x3d.shape
    bm = 512

    def body(s_ref, x_ref, o_ref):
        s = s_ref[0, 0]
        si = 1.0 / s
        q = jnp.clip(jnp.round(x_ref[...] * si), -Q_MAX, Q_MAX)
        o_ref[...] = q * s

    return pl.pallas_call(
        body,
        grid=(a, m // bm),
        in_specs=[
            pl.BlockSpec(memory_space=pltpu.SMEM),
            pl.BlockSpec((1, bm, w), lambda i, j: (i, j, 0)),
        ],
        out_specs=pl.BlockSpec((1, bm, w), lambda i, j: (i, j, 0)),
        out_shape=jax.ShapeDtypeStruct((a, m, w), jnp.float32),
    )(scale, x3d)


def kernel(x, gamma):
    n = x.size
    k = round(QUANTILE * n)
    x_flat = x.reshape(-1)
    x_bits = lax.bitcast_convert_type(x_flat, jnp.int32)

    h1 = _sc_hist(x_bits, None)
    b2d, r2d = _tc_reduce1(h1.reshape(NW, 512, 128), k)
    h2 = _sc_hist(x_bits, b2d.reshape(L))
    scale = _tc_reduce2(
        h2.reshape(NW, 512, 128), b2d, r2d, gamma.reshape(1, 1)
    )
    return _tc_quantize(x, scale)
